# baseline (speedup 1.0000x reference)
import functools

import jax
import jax.numpy as jnp
from jax import lax
from jax.experimental import pallas as pl
from jax.experimental.pallas import tpu as pltpu
from jax.experimental.pallas import tpu_sc as plsc

N_ATOMS = 100000
N_IN = 128
N_HIDDEN = 64
N_MOL = 1024

LANES = 16
NWORKERS = 32

HALF0 = 69632
BLK0 = 17408
HALF1 = 32768
BLK1 = 4096
SPARE = 8


def _mlp_body(x_ref, w1t_ref, b1_ref, w2_ref, b2_ref, y_ref, *, blk, base_row):
    i = pl.program_id(0)
    h = lax.dot_general(
        x_ref[...], w1t_ref[...], (((1,), (1,)), ((), ())),
        preferred_element_type=jnp.float32,
    )
    h = h + b1_ref[...]
    h = h * jax.nn.sigmoid(h)
    y = jnp.dot(h, w2_ref[...], preferred_element_type=jnp.float32) + b2_ref[...]
    rows_blk = blk // 128
    yw = y.reshape(rows_blk, 128)
    rows = (
        base_row + i * blk
        + lax.broadcasted_iota(jnp.int32, (rows_blk, 128), 0) * 128
        + lax.broadcasted_iota(jnp.int32, (rows_blk, 128), 1)
    )
    y_ref[...] = jnp.where(rows < N_ATOMS, yw, 0.0)


def _mlp_phase(x, W1t, b1r, W2, b2r, blk, grid, base_row):
    blk_off = base_row // blk
    rows_blk = blk // 128
    return pl.pallas_call(
        functools.partial(_mlp_body, blk=blk, base_row=base_row),
        grid=(grid,),
        in_specs=[
            pl.BlockSpec((blk, N_IN), lambda i, o=blk_off: (i + o, 0)),
            pl.BlockSpec((N_HIDDEN, N_IN), lambda i: (0, 0)),
            pl.BlockSpec((1, N_HIDDEN), lambda i: (0, 0)),
            pl.BlockSpec((N_HIDDEN, 1), lambda i: (0, 0)),
            pl.BlockSpec((1, 1), lambda i: (0, 0)),
        ],
        out_specs=pl.BlockSpec((rows_blk, 128), lambda i: (i, 0)),
        out_shape=jax.ShapeDtypeStruct(
            (grid * rows_blk + SPARE, 128), jnp.float32
        ),
    )(x, W1t, b1r, W2, b2r)


def _sc_segment_sum(y_wide, idx, base_atom, natoms_phase):
    chunk = natoms_phase // NWORKERS
    crows = chunk // 128
    window = ((crows + 7 + 7) // 8) * 8
    real = max(0, min(natoms_phase, N_ATOMS - base_atom))
    full = real // chunk
    part = real - full * chunk
    assert part % LANES == 0 and chunk % LANES == 0 and base_atom % 8 == 0

    mesh = plsc.VectorSubcoreMesh(core_axis_name="c", subcore_axis_name="s")

    @functools.partial(
        pl.kernel,
        mesh=mesh,
        out_type=jax.ShapeDtypeStruct((NWORKERS, N_MOL), jnp.float32),
        scratch_types=[
            pltpu.VMEM((window, 128), jnp.float32),
            pltpu.VMEM((chunk,), jnp.int32),
            pltpu.VMEM((LANES * N_MOL,), jnp.float32),
            pltpu.VMEM((N_MOL,), jnp.float32),
        ],
        compiler_params=pltpu.CompilerParams(needs_layout_passes=False),
    )
    def body(y_hbm, idx_hbm, out_hbm, y_v, idx_v, acc_v, row_v):
        wid = lax.axis_index("s") * 2 + lax.axis_index("c")
        row0 = wid * crows
        base8 = (row0 // 8) * 8
        delta = row0 - base8
        pltpu.sync_copy(y_hbm.at[pl.ds(base8, window)], y_v)

        zero_i = jnp.zeros((LANES,), jnp.int32)

        if full < NWORKERS:
            @pl.when(wid < full)
            def _():
                pltpu.sync_copy(
                    idx_hbm.at[pl.ds(base_atom + wid * chunk, chunk)], idx_v
                )

            if part > 0:
                @pl.when(wid == full)
                def _():
                    def zpad_body(k, _):
                        idx_v[pl.ds(part + k * LANES, LANES)] = zero_i
                        return 0

                    lax.fori_loop(0, (chunk - part) // LANES, zpad_body, 0)
                    pltpu.sync_copy(
                        idx_hbm.at[pl.ds(base_atom + full * chunk, part)],
                        idx_v.at[pl.ds(0, part)],
                    )

            @pl.when(wid > full)
            def _():
                def zall_body(k, _):
                    idx_v[pl.ds(k * LANES, LANES)] = zero_i
                    return 0

                lax.fori_loop(0, chunk // LANES, zall_body, 0)
        else:
            pltpu.sync_copy(
                idx_hbm.at[pl.ds(base_atom + wid * chunk, chunk)], idx_v
            )

        zero = jnp.zeros((LANES,), jnp.float32)

        def zero_body(k, _):
            for c in range(8):
                acc_v[pl.ds((k * 8 + c) * LANES, LANES)] = zero
            return 0

        lax.fori_loop(0, (LANES * N_MOL) // (8 * LANES), zero_body, 0)

        lane_off = lax.iota(jnp.int32, LANES) * N_MOL

        def row_body(r, _):
            for c in range(128 // LANES):
                idx = idx_v[pl.ds(r * 128 + c * LANES, LANES)] + lane_off
                val = y_v[delta + r, pl.ds(c * LANES, LANES)]
                plsc.addupdate_scatter(acc_v, [idx], val)
            return 0

        lax.fori_loop(0, crows, row_body, 0)

        def red_body(g, _):
            s = acc_v[pl.ds(g * LANES, LANES)]
            for r in range(1, LANES):
                s = s + acc_v[pl.ds(r * N_MOL + g * LANES, LANES)]
            row_v[pl.ds(g * LANES, LANES)] = s
            return 0

        lax.fori_loop(0, N_MOL // LANES, red_body, 0)
        pltpu.sync_copy(row_v, out_hbm.at[wid])

    return body(y_wide, idx)


def _combine_body(p0_ref, p1_ref, o_ref):
    o_ref[...] = jnp.sum(p0_ref[...], axis=0, keepdims=True) + jnp.sum(
        p1_ref[...], axis=0, keepdims=True
    )


def _combine(p0, p1):
    return pl.pallas_call(
        _combine_body,
        out_shape=jax.ShapeDtypeStruct((1, N_MOL), jnp.float32),
    )(p0, p1)


def kernel(scalar_representation, idx_m, W1, b1, W2, b2):
    W1t = W1.T
    b1r = b1.reshape(1, N_HIDDEN)
    b2r = b2.reshape(1, 1)
    idx = idx_m.astype(jnp.int32)
    x = scalar_representation

    y0 = _mlp_phase(x, W1t, b1r, W2, b2r, BLK0, HALF0 // BLK0, 0)
    p0 = _sc_segment_sum(y0, idx, 0, HALF0)
    y1 = _mlp_phase(x, W1t, b1r, W2, b2r, BLK1, HALF1 // BLK1, HALF0)
    p1 = _sc_segment_sum(y1, idx, HALF0, HALF1)
    out = _combine(p0, p1)
    return out.reshape(N_MOL)


# speedup vs baseline: 1.1024x; 1.1024x over previous
import functools

import jax
import jax.numpy as jnp
from jax import lax
from jax.experimental import pallas as pl
from jax.experimental.pallas import tpu as pltpu
from jax.experimental.pallas import tpu_sc as plsc

N_ATOMS = 100000
N_IN = 128
N_HIDDEN = 64
N_MOL = 1024

LANES = 16
NWORKERS = 32
BLK = 20480
GRID = 5
N_PAD = GRID * BLK
CHUNK = N_PAD // NWORKERS
NVECS = CHUNK // LANES
TAIL = N_ATOMS - (NWORKERS - 1) * CHUNK
CROWS = CHUNK // 128


ROWS = BLK // 128


def _mlp_body(x_ref, w1t_ref, b1_ref, w2_ref, b2_ref, y_ref):
    i = pl.program_id(0)
    h = lax.dot_general(
        x_ref[...], w1t_ref[...], (((1,), (1,)), ((), ())),
        preferred_element_type=jnp.float32,
    )
    h = h + b1_ref[...]
    h = h * jax.nn.sigmoid(h)
    y = jnp.dot(h, w2_ref[...], preferred_element_type=jnp.float32) + b2_ref[...]
    yw = y.reshape(ROWS, 128)
    rows = (
        i * BLK
        + lax.broadcasted_iota(jnp.int32, (ROWS, 128), 0) * 128
        + lax.broadcasted_iota(jnp.int32, (ROWS, 128), 1)
    )
    y_ref[...] = jnp.where(rows < N_ATOMS, yw, 0.0)


def _mlp(x, W1, b1, W2, b2):
    return pl.pallas_call(
        _mlp_body,
        grid=(GRID,),
        in_specs=[
            pl.BlockSpec((BLK, N_IN), lambda i: (i, 0)),
            pl.BlockSpec((N_HIDDEN, N_IN), lambda i: (0, 0)),
            pl.BlockSpec((1, N_HIDDEN), lambda i: (0, 0)),
            pl.BlockSpec((N_HIDDEN, 1), lambda i: (0, 0)),
            pl.BlockSpec((1, 1), lambda i: (0, 0)),
        ],
        out_specs=pl.BlockSpec((ROWS, 128), lambda i: (i, 0)),
        out_shape=jax.ShapeDtypeStruct((N_PAD // 128, 128), jnp.float32),
    )(x, W1.T, b1.reshape(1, N_HIDDEN), W2, b2.reshape(1, 1))


def _sc_segment_sum(y_wide, idx_pad):
    mesh = plsc.VectorSubcoreMesh(core_axis_name="c", subcore_axis_name="s")

    @functools.partial(
        pl.kernel,
        mesh=mesh,
        out_type=jax.ShapeDtypeStruct((NWORKERS, N_MOL), jnp.float32),
        scratch_types=[
            pltpu.VMEM((CROWS + 7, 128), jnp.float32),
            pltpu.VMEM((CHUNK,), jnp.int32),
            pltpu.VMEM((N_MOL,), jnp.float32),
        ],
        compiler_params=pltpu.CompilerParams(needs_layout_passes=False),
    )
    def body(y_hbm, idx_hbm, out_hbm, y_v, idx_v, acc_v):
        wid = lax.axis_index("s") * 2 + lax.axis_index("c")
        row0 = wid * CROWS
        base8 = (row0 // 8) * 8
        delta = row0 - base8
        pltpu.sync_copy(y_hbm.at[pl.ds(base8, CROWS + 7)], y_v)

        zero_i = jnp.zeros((LANES,), jnp.int32)

        @pl.when(wid < NWORKERS - 1)
        def _():
            pltpu.sync_copy(idx_hbm.at[pl.ds(wid * CHUNK, CHUNK)], idx_v)

        @pl.when(wid == NWORKERS - 1)
        def _():
            def zpad_body(k, _):
                idx_v[pl.ds(TAIL + k * LANES, LANES)] = zero_i
                return 0

            lax.fori_loop(0, (CHUNK - TAIL) // LANES, zpad_body, 0)
            pltpu.sync_copy(
                idx_hbm.at[pl.ds((NWORKERS - 1) * CHUNK, TAIL)],
                idx_v.at[pl.ds(0, TAIL)],
            )

        zero = jnp.zeros((LANES,), jnp.float32)

        def zero_body(k, _):
            acc_v[pl.ds(k * LANES, LANES)] = zero
            return 0

        lax.fori_loop(0, N_MOL // LANES, zero_body, 0)

        @plsc.parallel_loop(0, CROWS, 1, unroll=4)
        def row_body(r):
            for c in range(128 // LANES):
                idx = idx_v[pl.ds(r * 128 + c * LANES, LANES)]
                val = y_v[delta + r, pl.ds(c * LANES, LANES)]
                plsc.addupdate_scatter(acc_v, [idx], val)

        pltpu.sync_copy(acc_v, out_hbm.at[wid])

    return body(y_wide, idx_pad)


def _combine_body(p_ref, o_ref):
    o_ref[...] = jnp.sum(p_ref[...], axis=0, keepdims=True)


def _combine(partials):
    return pl.pallas_call(
        _combine_body,
        out_shape=jax.ShapeDtypeStruct((1, N_MOL), jnp.float32),
    )(partials)


def kernel(scalar_representation, idx_m, W1, b1, W2, b2):
    y_wide = _mlp(scalar_representation, W1, b1, W2, b2)
    partials = _sc_segment_sum(y_wide, idx_m.astype(jnp.int32))
    out = _combine(partials)
    return out.reshape(N_MOL)


# speedup vs baseline: 1.1308x; 1.0258x over previous
import functools

import jax
import jax.numpy as jnp
from jax import lax
from jax.experimental import pallas as pl
from jax.experimental.pallas import tpu as pltpu
from jax.experimental.pallas import tpu_sc as plsc

N_ATOMS = 100000
N_IN = 128
N_HIDDEN = 64
N_MOL = 1024

LANES = 16
NWORKERS = 32
BLK = 20480
GRID = 5
N_PAD = GRID * BLK
CHUNK = N_PAD // NWORKERS
NVECS = CHUNK // LANES
TAIL = N_ATOMS - (NWORKERS - 1) * CHUNK
CROWS = CHUNK // 128


ROWS = BLK // 128


def _mlp_body(x_ref, w1t_ref, b1_ref, w2_ref, b2_ref, y_ref):
    i = pl.program_id(0)
    h = lax.dot_general(
        x_ref[...], w1t_ref[...], (((1,), (1,)), ((), ())),
        preferred_element_type=jnp.float32,
    )
    h = h + b1_ref[...]
    h = h * jax.nn.sigmoid(h)
    y = jnp.dot(h, w2_ref[...], preferred_element_type=jnp.float32) + b2_ref[...]
    yw = y.reshape(ROWS, 128)
    rows = (
        i * BLK
        + lax.broadcasted_iota(jnp.int32, (ROWS, 128), 0) * 128
        + lax.broadcasted_iota(jnp.int32, (ROWS, 128), 1)
    )
    y_ref[...] = jnp.where(rows < N_ATOMS, yw, 0.0)


def _mlp(x, W1, b1, W2, b2):
    return pl.pallas_call(
        _mlp_body,
        grid=(GRID,),
        in_specs=[
            pl.BlockSpec((BLK, N_IN), lambda i: (i, 0)),
            pl.BlockSpec((N_HIDDEN, N_IN), lambda i: (0, 0)),
            pl.BlockSpec((1, N_HIDDEN), lambda i: (0, 0)),
            pl.BlockSpec((N_HIDDEN, 1), lambda i: (0, 0)),
            pl.BlockSpec((1, 1), lambda i: (0, 0)),
        ],
        out_specs=pl.BlockSpec((ROWS, 128), lambda i: (i, 0)),
        out_shape=jax.ShapeDtypeStruct((N_PAD // 128, 128), jnp.float32),
    )(x, W1.T, b1.reshape(1, N_HIDDEN), W2, b2.reshape(1, 1))


def _sc_segment_sum(y_wide, idx_pad):
    mesh = plsc.VectorSubcoreMesh(core_axis_name="c", subcore_axis_name="s")

    @functools.partial(
        pl.kernel,
        mesh=mesh,
        out_type=jax.ShapeDtypeStruct((NWORKERS, N_MOL), jnp.float32),
        scratch_types=[
            pltpu.VMEM((CROWS + 7, 128), jnp.float32),
            pltpu.VMEM((CHUNK,), jnp.int32),
            pltpu.VMEM((N_MOL,), jnp.float32),
            pltpu.SemaphoreType.DMA,
            pltpu.SemaphoreType.DMA,
        ],
        compiler_params=pltpu.CompilerParams(needs_layout_passes=False),
    )
    def body(y_hbm, idx_hbm, out_hbm, y_v, idx_v, acc_v, sem_y, sem_i):
        wid = lax.axis_index("s") * 2 + lax.axis_index("c")
        row0 = wid * CROWS
        base8 = (row0 // 8) * 8
        delta = row0 - base8
        cp_y = pltpu.async_copy(y_hbm.at[pl.ds(base8, CROWS + 7)], y_v, sem_y)

        zero_i = jnp.zeros((LANES,), jnp.int32)

        @pl.when(wid < NWORKERS - 1)
        def _():
            pltpu.async_copy(
                idx_hbm.at[pl.ds(wid * CHUNK, CHUNK)], idx_v, sem_i
            ).wait()

        @pl.when(wid == NWORKERS - 1)
        def _():
            cp_i = pltpu.async_copy(
                idx_hbm.at[pl.ds((NWORKERS - 1) * CHUNK, TAIL)],
                idx_v.at[pl.ds(0, TAIL)],
                sem_i,
            )

            def zpad_body(k, _):
                idx_v[pl.ds(TAIL + k * LANES, LANES)] = zero_i
                return 0

            lax.fori_loop(0, (CHUNK - TAIL) // LANES, zpad_body, 0)
            cp_i.wait()

        zero = jnp.zeros((LANES,), jnp.float32)

        @plsc.parallel_loop(0, N_MOL // LANES, 1, unroll=8)
        def zero_body(k):
            acc_v[pl.ds(k * LANES, LANES)] = zero

        cp_y.wait()

        @plsc.parallel_loop(0, CROWS, 1, unroll=4)
        def row_body(r):
            for c in range(128 // LANES):
                idx = idx_v[pl.ds(r * 128 + c * LANES, LANES)]
                val = y_v[delta + r, pl.ds(c * LANES, LANES)]
                plsc.addupdate_scatter(acc_v, [idx], val)

        pltpu.sync_copy(acc_v, out_hbm.at[wid])

    return body(y_wide, idx_pad)


def _combine_body(p_ref, o_ref):
    o_ref[...] = jnp.sum(p_ref[...], axis=0, keepdims=True)


def _combine(partials):
    return pl.pallas_call(
        _combine_body,
        out_shape=jax.ShapeDtypeStruct((1, N_MOL), jnp.float32),
    )(partials)


def kernel(scalar_representation, idx_m, W1, b1, W2, b2):
    y_wide = _mlp(scalar_representation, W1, b1, W2, b2)
    partials = _sc_segment_sum(y_wide, idx_m.astype(jnp.int32))
    out = _combine(partials)
    return out.reshape(N_MOL)
